# Initial kernel scaffold; baseline (speedup 1.0000x reference)
import jax
import jax.numpy as jnp
from jax.experimental import pallas as pl


def kernel(nodes, neigh_idx, feat_table, W1, b1):
    raise NotImplementedError("write your pallas kernel here")


# speedup vs baseline: 1.3241x; 1.3241x over previous
import functools

import jax
import jax.numpy as jnp
from jax import lax
from jax.experimental import pallas as pl
from jax.experimental.pallas import tpu as pltpu
from jax.experimental.pallas import tpu_sc as plsc

_NUM_CORES = 2
_NUM_SUBCORES = 16
_NW = _NUM_CORES * _NUM_SUBCORES
_LANES = 16

_C = 8
_SUB = 3


def _sc_gather_concat_mean(idx_flat, feat_table, bp, kp, d):
    npw = bp // _NW
    nchunk = npw // _C
    ipc = _C * kp
    subn = ipc // _SUB
    lg = d // _LANES
    inv_k = jnp.float32(1.0 / (kp - 1))

    assert npw * _NW == bp and nchunk * _C == npw and nchunk % 2 == 0
    assert subn * _SUB == ipc and subn % 8 == 0 and subn <= 128
    assert lg * _LANES == d

    mesh = plsc.VectorSubcoreMesh(
        core_axis_name="c", subcore_axis_name="s",
        num_cores=_NUM_CORES, num_subcores=_NUM_SUBCORES)

    @functools.partial(
        pl.kernel,
        mesh=mesh,
        out_type=jax.ShapeDtypeStruct((bp, 2 * d), jnp.float32),
        scratch_types=[
            pltpu.VMEM((ipc,), jnp.int32),
            pltpu.VMEM((ipc,), jnp.int32),
            pltpu.VMEM((ipc, d), jnp.float32),
            pltpu.VMEM((ipc, d), jnp.float32),
            pltpu.VMEM((_C, 2 * d), jnp.float32),
            pltpu.VMEM((_C, 2 * d), jnp.float32),
            pltpu.SemaphoreType.DMA,
            pltpu.SemaphoreType.DMA,
            pltpu.SemaphoreType.DMA,
            pltpu.SemaphoreType.DMA,
            pltpu.SemaphoreType.DMA,
            pltpu.SemaphoreType.DMA,
        ],
    )
    def k(table_hbm, idx_hbm, out_hbm, idx_v0, idx_v1, rows_v0, rows_v1,
          stage_v0, stage_v1, si0, si1, sr0, sr1, so0, so1):
        idx_v = (idx_v0, idx_v1)
        rows_v = (rows_v0, rows_v1)
        stage_v = (stage_v0, stage_v1)
        si = (si0, si1)
        sr = (sr0, sr1)
        so = (so0, so1)
        wid = lax.axis_index("s") * _NUM_CORES + lax.axis_index("c")
        woff = wid * (npw * kp)
        nbase = wid * npw

        def idx_copy(chunk, buf):
            return pltpu.make_async_copy(
                idx_hbm.at[pl.ds(woff + chunk * ipc, ipc)],
                idx_v[buf], si[buf])

        def gather_copy(buf, s):
            return pltpu.make_async_copy(
                table_hbm.at[idx_v[buf].at[pl.ds(s * subn, subn)]],
                rows_v[buf].at[pl.ds(s * subn, subn)], sr[buf])

        def out_copy(chunk, buf):
            return pltpu.make_async_copy(
                stage_v[buf],
                out_hbm.at[pl.ds(nbase + chunk * _C, _C)], so[buf])

        pltpu.sync_copy(idx_hbm.at[pl.ds(woff, ipc)], idx_v[0])
        for s in range(_SUB):
            gather_copy(0, s).start()
        idx_copy(1, 1).start()

        def reduce_chunk(buf):
            rows = rows_v[buf]
            stage = stage_v[buf]

            def node_body(j, carry):
                rb = j * kp
                for g in range(lg):
                    sl = pl.ds(g * _LANES, _LANES)
                    stage[j, sl] = rows[rb, sl]
                    acc = rows[rb + 1, sl]
                    for kk in range(2, kp):
                        acc = acc + rows[rb + kk, sl]
                    stage[j, pl.ds(d + g * _LANES, _LANES)] = acc * inv_k
                return carry
            lax.fori_loop(0, _C, node_body, 0, unroll=False)

        def loop_body(c2, carry):
            cc0 = c2 * 2
            for b in range(2):
                cc = cc0 + b
                nxt = cc + 1
                ob = 1 - b

                def launch_next():
                    idx_copy(nxt, ob).wait()
                    for s in range(_SUB):
                        gather_copy(ob, s).start()
                if b == 0:
                    launch_next()
                else:
                    @pl.when(nxt < nchunk)
                    def _():
                        launch_next()

                for s in range(_SUB):
                    gather_copy(b, s).wait()

                @pl.when(cc + 2 < nchunk)
                def _():
                    idx_copy(cc + 2, b).start()

                @pl.when(cc >= 2)
                def _():
                    out_copy(cc - 2, b).wait()

                reduce_chunk(b)
                out_copy(cc, b).start()
            return carry

        lax.fori_loop(0, nchunk // 2, loop_body, 0, unroll=False)
        out_copy(nchunk - 2, 0).wait()
        out_copy(nchunk - 1, 1).wait()

    return k(feat_table, idx_flat)


def _linear_body(x_ref, w_ref, b_ref, o_ref):
    acc = jnp.dot(x_ref[...], w_ref[...], preferred_element_type=jnp.float32)
    o_ref[...] = jnp.maximum(acc + b_ref[...], 0.0)


def _tc_linear(x, w1, b1):
    bp, d2 = x.shape
    d = w1.shape[1]
    tb = 1024
    assert bp % tb == 0
    return pl.pallas_call(
        _linear_body,
        grid=(bp // tb,),
        in_specs=[
            pl.BlockSpec((tb, d2), lambda i: (i, 0)),
            pl.BlockSpec((d2, d), lambda i: (0, 0)),
            pl.BlockSpec((1, d), lambda i: (0, 0)),
        ],
        out_specs=pl.BlockSpec((tb, d), lambda i: (i, 0)),
        out_shape=jax.ShapeDtypeStruct((bp, d), jnp.float32),
    )(x, w1, b1.reshape(1, d))


def kernel(nodes, neigh_idx, feat_table, W1, b1):
    b, k = neigh_idx.shape
    d = feat_table.shape[1]
    kp = k + 1

    bp = -(-b // (_NW * _C * 2)) * (_NW * _C * 2)
    idx = jnp.concatenate(
        [nodes.astype(jnp.int32).reshape(b, 1), neigh_idx.astype(jnp.int32)],
        axis=1)
    idx = jnp.concatenate([idx, jnp.zeros((bp - b, kp), jnp.int32)], axis=0)

    comb = _sc_gather_concat_mean(idx.reshape(-1), feat_table, bp, kp, d)
    out = _tc_linear(comb, W1, b1)
    return out[:b]


# speedup vs baseline: 1.4125x; 1.0668x over previous
import functools

import jax
import jax.numpy as jnp
from jax import lax
from jax.experimental import pallas as pl
from jax.experimental.pallas import tpu as pltpu
from jax.experimental.pallas import tpu_sc as plsc

_NUM_CORES = 2
_NUM_SUBCORES = 16
_NW = _NUM_CORES * _NUM_SUBCORES
_LANES = 16

_C = 8
_SUB = 3


def _sc_gather_concat_mean(idx_flat, feat_table, bp, kp, d, frac0):
    gran = _NUM_SUBCORES * _C * 2
    n0 = int(round(bp * frac0 / gran)) * gran
    n1 = bp - n0
    npw0 = n0 // _NUM_SUBCORES
    npw1 = n1 // _NUM_SUBCORES
    nchunk0 = npw0 // _C
    nchunk1 = npw1 // _C
    ipc = _C * kp
    subn = ipc // _SUB
    lg = d // _LANES
    inv_k = jnp.float32(1.0 / (kp - 1))
    nchunk_max = max(nchunk0, nchunk1)

    assert npw0 * _NUM_SUBCORES == n0 and npw1 * _NUM_SUBCORES == n1
    assert nchunk0 * _C == npw0 and nchunk0 % 2 == 0
    assert nchunk1 * _C == npw1 and nchunk1 % 2 == 0
    assert subn * _SUB == ipc and subn % 8 == 0 and subn <= 128
    assert lg * _LANES == d

    mesh = plsc.VectorSubcoreMesh(
        core_axis_name="c", subcore_axis_name="s",
        num_cores=_NUM_CORES, num_subcores=_NUM_SUBCORES)

    @functools.partial(
        pl.kernel,
        mesh=mesh,
        out_type=jax.ShapeDtypeStruct((bp, 2 * d), jnp.float32),
        scratch_types=[
            pltpu.VMEM((ipc,), jnp.int32),
            pltpu.VMEM((ipc,), jnp.int32),
            pltpu.VMEM((ipc, d), jnp.float32),
            pltpu.VMEM((ipc, d), jnp.float32),
            pltpu.VMEM((_C, 2 * d), jnp.float32),
            pltpu.VMEM((_C, 2 * d), jnp.float32),
            pltpu.SemaphoreType.DMA,
            pltpu.SemaphoreType.DMA,
            pltpu.SemaphoreType.DMA,
            pltpu.SemaphoreType.DMA,
            pltpu.SemaphoreType.DMA,
            pltpu.SemaphoreType.DMA,
        ],
    )
    def k(table_hbm, idx_hbm, out_hbm, idx_v0, idx_v1, rows_v0, rows_v1,
          stage_v0, stage_v1, si0, si1, sr0, sr1, so0, so1):
        idx_v = (idx_v0, idx_v1)
        rows_v = (rows_v0, rows_v1)
        stage_v = (stage_v0, stage_v1)
        si = (si0, si1)
        sr = (sr0, sr1)
        so = (so0, so1)
        cid = lax.axis_index("c")
        sid = lax.axis_index("s")
        nbase = jnp.where(cid == 0, sid * npw0, n0 + sid * npw1)
        nchunk = jnp.where(cid == 0, nchunk0, nchunk1)
        woff = nbase * kp

        def idx_copy(chunk, buf):
            return pltpu.make_async_copy(
                idx_hbm.at[pl.ds(woff + chunk * ipc, ipc)],
                idx_v[buf], si[buf])

        def gather_copy(buf, s):
            return pltpu.make_async_copy(
                table_hbm.at[idx_v[buf].at[pl.ds(s * subn, subn)]],
                rows_v[buf].at[pl.ds(s * subn, subn)], sr[buf])

        def out_copy(chunk, buf):
            return pltpu.make_async_copy(
                stage_v[buf],
                out_hbm.at[pl.ds(nbase + chunk * _C, _C)], so[buf])

        pltpu.sync_copy(idx_hbm.at[pl.ds(woff, ipc)], idx_v[0])
        for s in range(_SUB):
            gather_copy(0, s).start()
        idx_copy(1, 1).start()

        def reduce_chunk(buf):
            rows = rows_v[buf]
            stage = stage_v[buf]

            def node_body(j, carry):
                rb = j * kp
                for g in range(lg):
                    sl = pl.ds(g * _LANES, _LANES)
                    stage[j, sl] = rows[rb, sl]
                    acc = rows[rb + 1, sl]
                    for kk in range(2, kp):
                        acc = acc + rows[rb + kk, sl]
                    stage[j, pl.ds(d + g * _LANES, _LANES)] = acc * inv_k
                return carry
            lax.fori_loop(0, _C, node_body, 0, unroll=False)

        def loop_body(c2, carry):
            cc0 = c2 * 2
            for b in range(2):
                cc = cc0 + b
                nxt = cc + 1
                ob = 1 - b

                def launch_next():
                    idx_copy(nxt, ob).wait()
                    for s in range(_SUB):
                        gather_copy(ob, s).start()
                if b == 0:
                    launch_next()
                else:
                    @pl.when(nxt < nchunk)
                    def _():
                        launch_next()

                for s in range(_SUB):
                    gather_copy(b, s).wait()

                @pl.when(cc + 2 < nchunk)
                def _():
                    idx_copy(cc + 2, b).start()

                @pl.when(cc >= 2)
                def _():
                    out_copy(cc - 2, b).wait()

                reduce_chunk(b)
                out_copy(cc, b).start()
            return carry

        lax.fori_loop(0, nchunk // 2, loop_body, 0, unroll=False)
        out_copy(nchunk - 2, 0).wait()
        out_copy(nchunk - 1, 1).wait()

    return k(feat_table, idx_flat)


def _linear_body(x_ref, w_ref, b_ref, o_ref):
    acc = jnp.dot(x_ref[...], w_ref[...], preferred_element_type=jnp.float32)
    o_ref[...] = jnp.maximum(acc + b_ref[...], 0.0)


def _tc_linear(x, w1, b1):
    bp, d2 = x.shape
    d = w1.shape[1]
    tb = 1024
    assert bp % tb == 0
    return pl.pallas_call(
        _linear_body,
        grid=(bp // tb,),
        in_specs=[
            pl.BlockSpec((tb, d2), lambda i: (i, 0)),
            pl.BlockSpec((d2, d), lambda i: (0, 0)),
            pl.BlockSpec((1, d), lambda i: (0, 0)),
        ],
        out_specs=pl.BlockSpec((tb, d), lambda i: (i, 0)),
        out_shape=jax.ShapeDtypeStruct((bp, d), jnp.float32),
    )(x, w1, b1.reshape(1, d))


def kernel(nodes, neigh_idx, feat_table, W1, b1):
    b, k = neigh_idx.shape
    d = feat_table.shape[1]
    kp = k + 1

    bp = -(-b // (_NW * _C * 2)) * (_NW * _C * 2)
    idx = jnp.concatenate(
        [nodes.astype(jnp.int32).reshape(b, 1), neigh_idx.astype(jnp.int32)],
        axis=1)
    idx = jnp.concatenate([idx, jnp.zeros((bp - b, kp), jnp.int32)], axis=0)

    comb = _sc_gather_concat_mean(idx.reshape(-1), feat_table, bp, kp, d,
                                  frac0=0.8)
    out = _tc_linear(comb, W1, b1)
    return out[:b]
